# Initial kernel scaffold; baseline (speedup 1.0000x reference)
import jax
import jax.numpy as jnp
from jax.experimental import pallas as pl


def kernel(pre_normal_scores, oh_att, tf_att, anchors, variances, select_normals, select_abnormals):
    raise NotImplementedError("write your pallas kernel here")


# speedup vs baseline: 7.5161x; 7.5161x over previous
import functools

import jax
import jax.numpy as jnp
from jax import lax
from jax.experimental import pallas as pl
from jax.experimental.pallas import tpu as pltpu
from jax.experimental.pallas import tpu_sc as plsc

_L, _K, _C = 2, 2048, 1024
_B, _T = 64, 4096
_BK = 512
_NC, _NS, _LANES = 2, 16, 16
_RPT = 8
_NTPA = _B // _RPT
_UNROLL = 4


def _lane_reduce(v, op):
    idx = lax.iota(jnp.int32, _LANES)
    for s in (8, 4, 2, 1):
        v = op(v, v.at[idx ^ s].get(mode="promise_in_bounds"))
    return v


def _row_reduce(buf, r, with_sq):
    def body(i, carry):
        ms, ss = carry
        new_ms, new_ss = [], []
        for u in range(_UNROLL):
            v = buf[r, pl.ds((i * _UNROLL + u) * _LANES, _LANES)]
            new_ms.append(jnp.maximum(ms[u], v))
            if with_sq:
                new_ss.append(ss[u] + v * v)
        return tuple(new_ms), (tuple(new_ss) if with_sq else ss)

    neg = jnp.full((_LANES,), -jnp.inf, jnp.float32)
    zero = jnp.zeros((_LANES,), jnp.float32)
    m0 = (neg,) * _UNROLL
    s0 = (zero,) * _UNROLL if with_sq else ()
    m, s = lax.fori_loop(0, _T // (_UNROLL * _LANES), body, (m0, s0))
    mv = jnp.maximum(jnp.maximum(m[0], m[1]), jnp.maximum(m[2], m[3]))
    rmax = _lane_reduce(mv, jnp.maximum)
    if with_sq:
        rsq = _lane_reduce((s[0] + s[1]) + (s[2] + s[3]), jnp.add)
        return rmax, rsq
    return rmax, None


def _sc_rowstats_body(p_hbm, oh_hbm, tf_hbm,
                      an_out, ohm_out, tfm_out, ssq_out,
                      buf, outa, outb):
    wid = lax.axis_index("s") * _NC + lax.axis_index("c")

    idx = lax.iota(jnp.int32, _LANES)

    @pl.when(wid < _NTPA)
    def _pre_tasks():
        base = wid * _RPT
        pltpu.sync_copy(p_hbm.at[pl.ds(base, _RPT)], buf)
        accm = jnp.zeros((_LANES,), jnp.float32)
        accs = jnp.zeros((_LANES,), jnp.float32)
        for r in range(_RPT):
            rmax, rsq = _row_reduce(buf, r, True)
            accm = jnp.where(idx == r, rmax, accm)
            accs = jnp.where(idx == r, rsq, accs)
        outa[...] = accm
        outb[...] = accs
        pltpu.sync_copy(outa.at[pl.ds(0, _RPT)], an_out.at[pl.ds(base, _RPT)])
        pltpu.sync_copy(outb.at[pl.ds(0, _RPT)], ssq_out.at[pl.ds(base, _RPT)])

    @pl.when(jnp.logical_and(wid >= _NTPA, wid < 2 * _NTPA))
    def _oh_tasks():
        base = (wid - _NTPA) * _RPT
        pltpu.sync_copy(oh_hbm.at[pl.ds(base, _RPT)], buf)
        accm = jnp.zeros((_LANES,), jnp.float32)
        for r in range(_RPT):
            rmax, _ = _row_reduce(buf, r, False)
            accm = jnp.where(idx == r, rmax, accm)
        outa[...] = accm
        pltpu.sync_copy(outa.at[pl.ds(0, _RPT)], ohm_out.at[pl.ds(base, _RPT)])

    @pl.when(jnp.logical_and(wid >= 2 * _NTPA, wid < 3 * _NTPA))
    def _tf_tasks():
        base = (wid - 2 * _NTPA) * _RPT
        pltpu.sync_copy(tf_hbm.at[pl.ds(base, _RPT)], buf)
        accm = jnp.zeros((_LANES,), jnp.float32)
        for r in range(_RPT):
            rmax, _ = _row_reduce(buf, r, False)
            accm = jnp.where(idx == r, rmax, accm)
        outa[...] = accm
        pltpu.sync_copy(outa.at[pl.ds(0, _RPT)], tfm_out.at[pl.ds(base, _RPT)])


def _sqrt16(x):
    i = lax.bitcast_convert_type(x, jnp.int32)
    y = lax.bitcast_convert_type(
        jnp.int32(0x5F3759DF) - lax.shift_right_arithmetic(i, 1), jnp.float32)
    for _ in range(4):
        y = y * (1.5 - 0.5 * x * y * y)
    return jnp.where(x > 0.0, x * y, 0.0)


def _minmax64(vec_ref, scale):
    gmax = jnp.full((_LANES,), -jnp.inf, jnp.float32)
    gmin = jnp.full((_LANES,), jnp.inf, jnp.float32)
    for i in range(_B // _LANES):
        v = vec_ref[pl.ds(i * _LANES, _LANES)] * scale
        gmax = jnp.maximum(gmax, v)
        gmin = jnp.minimum(gmin, v)
    return _lane_reduce(gmax, jnp.maximum), _lane_reduce(gmin, jnp.minimum)


def _sc_combine_body(an_hbm, ohm_hbm, tfm_hbm, ssq_hbm, res_out,
                     an_v, ohm_v, tfm_v, ssq_v, res_v):
    wid = lax.axis_index("s") * _NC + lax.axis_index("c")

    @pl.when(wid == 0)
    def _combine():
        pltpu.sync_copy(an_hbm, an_v)
        pltpu.sync_copy(ohm_hbm, ohm_v)
        pltpu.sync_copy(tfm_hbm, tfm_v)
        pltpu.sync_copy(ssq_hbm, ssq_v)

        omax, omin = _minmax64(ohm_v, 1.0)
        tmax, tmin = _minmax64(tfm_v, 2.5)
        o_scale = jnp.where(omax > 1.0, 1.0 / (omax - omin), 1.0)
        o_shift = jnp.where(omax > 1.0, omin, 0.0)
        t_scale = jnp.where(tmax > 1.0, 1.0 / (tmax - tmin), 1.0)
        t_shift = jnp.where(tmax > 1.0, tmin, 0.0)

        acc_hp = jnp.zeros((_LANES,), jnp.float32)
        acc_n = jnp.zeros((_LANES,), jnp.float32)
        for i in range(_B // _LANES):
            sl = pl.ds(i * _LANES, _LANES)
            ohn = (ohm_v[sl] - o_shift) * o_scale
            tfn = (tfm_v[sl] * 2.5 - t_shift) * t_scale
            hp = jnp.maximum(ohn, tfn)
            d = hp - an_v[sl]
            acc_hp = acc_hp + d * d
            acc_n = acc_n + _sqrt16(ssq_v[sl])

        normal_loss = _lane_reduce(acc_n, jnp.add) / _B
        hp_loss = _lane_reduce(acc_hp, jnp.add) / _B
        idx = lax.iota(jnp.int32, 16)
        res = jnp.where(idx == 0, normal_loss,
                        jnp.where(idx == 1, hp_loss, 0.0))
        res_v[...] = res
        pltpu.sync_copy(res_v, res_out)


def _mpp_kernel(anchors_ref, variances_ref, sn_ref, sa_ref, out_ref, acc_ref):
    l = pl.program_id(0)
    kb = pl.program_id(1)

    @pl.when(jnp.logical_and(l == 0, kb == 0))
    def _init():
        acc_ref[0] = 0.0

    x = sn_ref[0]
    y = sa_ref[0]
    mu = anchors_ref[0]
    inv_var = 1.0 / variances_ref[0]
    dx = x - mu
    dy = y - mu
    d_pos = jnp.sqrt(jnp.sum(dx * dx * inv_var, axis=1, keepdims=True))
    d_neg = jnp.sqrt(jnp.sum(dy * dy * inv_var, axis=1, keepdims=True))
    acc_ref[0] += jnp.sum(jnp.maximum(d_pos - d_neg + 1.0, 0.0))

    @pl.when(jnp.logical_and(l == _L - 1, kb == _K // _BK - 1))
    def _finish():
        out_ref[0] = acc_ref[0] / _K


_f32_64 = jax.ShapeDtypeStruct((_B,), jnp.float32)

_sc_rowstats = functools.partial(
    pl.kernel,
    out_type=(_f32_64, _f32_64, _f32_64, _f32_64),
    mesh=plsc.VectorSubcoreMesh(core_axis_name="c", subcore_axis_name="s"),
    scratch_types=[
        pltpu.VMEM((_RPT, _T), jnp.float32),
        pltpu.VMEM((_LANES,), jnp.float32),
        pltpu.VMEM((_LANES,), jnp.float32),
    ],
)(_sc_rowstats_body)

_sc_combine = functools.partial(
    pl.kernel,
    out_type=jax.ShapeDtypeStruct((_LANES,), jnp.float32),
    mesh=plsc.VectorSubcoreMesh(core_axis_name="c", subcore_axis_name="s"),
    scratch_types=[
        pltpu.VMEM((_B,), jnp.float32),
        pltpu.VMEM((_B,), jnp.float32),
        pltpu.VMEM((_B,), jnp.float32),
        pltpu.VMEM((_B,), jnp.float32),
        pltpu.VMEM((_LANES,), jnp.float32),
    ],
)(_sc_combine_body)


def kernel(pre_normal_scores, oh_att, tf_att, anchors, variances,
           select_normals, select_abnormals):
    an, ohm, tfm, ssq = _sc_rowstats(pre_normal_scores, oh_att, tf_att)
    res = _sc_combine(an, ohm, tfm, ssq)

    mpp = pl.pallas_call(
        _mpp_kernel,
        grid=(_L, _K // _BK),
        in_specs=[
            pl.BlockSpec((1, 1, _C), lambda l, kb: (l, 0, 0)),
            pl.BlockSpec((1, 1, _C), lambda l, kb: (l, 0, 0)),
            pl.BlockSpec((1, _BK, _C), lambda l, kb: (l, kb, 0)),
            pl.BlockSpec((1, _BK, _C), lambda l, kb: (l, kb, 0)),
        ],
        out_specs=pl.BlockSpec(memory_space=pltpu.SMEM),
        out_shape=jax.ShapeDtypeStruct((1,), jnp.float32),
        scratch_shapes=[pltpu.SMEM((1,), jnp.float32)],
    )(anchors.reshape(_L, 1, _C), variances.reshape(_L, 1, _C),
      select_normals, select_abnormals)

    normal_loss = res[0]
    hp_loss = res[1]
    mpp_loss = mpp[0]
    total_loss = normal_loss + mpp_loss
    new_cost = 0.9 * total_loss + hp_loss
    return new_cost, normal_loss, mpp_loss, total_loss


# speedup vs baseline: 9.6643x; 1.2858x over previous
import functools

import jax
import jax.numpy as jnp
from jax import lax
from jax.experimental import pallas as pl
from jax.experimental.pallas import tpu as pltpu
from jax.experimental.pallas import tpu_sc as plsc

_L, _K, _C = 2, 2048, 1024
_B, _T = 64, 4096
_BK = 512
_NC, _NS, _LANES = 2, 16, 16
_RPT = 8
_NTPA = _B // _RPT
_UNROLL = 4


def _lane_reduce(v, op):
    idx = lax.iota(jnp.int32, _LANES)
    for s in (8, 4, 2, 1):
        v = op(v, v.at[idx ^ s].get(mode="promise_in_bounds"))
    return v


def _row_stats(buf, r, with_sq):
    def body(i, carry):
        ms, ss = carry
        new_ms, new_ss = [], []
        for u in range(_UNROLL):
            v = buf[r, pl.ds((i * _UNROLL + u) * _LANES, _LANES)]
            new_ms.append(jnp.maximum(ms[u], v))
            if with_sq:
                new_ss.append(ss[u] + v * v)
        return tuple(new_ms), (tuple(new_ss) if with_sq else ss)

    neg = jnp.full((_LANES,), -jnp.inf, jnp.float32)
    zero = jnp.zeros((_LANES,), jnp.float32)
    m0 = (neg,) * _UNROLL
    s0 = (zero,) * _UNROLL if with_sq else ()
    m, s = lax.fori_loop(0, _T // (_UNROLL * _LANES), body, (m0, s0))
    mv = jnp.maximum(jnp.maximum(m[0], m[1]), jnp.maximum(m[2], m[3]))
    rmax = _lane_reduce(mv, jnp.maximum)
    if with_sq:
        rsq = _lane_reduce((s[0] + s[1]) + (s[2] + s[3]), jnp.add)
        return rmax, rsq
    return rmax, None


def _sc_rowstats_body(p_hbm, oh_hbm, tf_hbm, stats_out, buf, outa, outb):
    wid = lax.axis_index("s") * _NC + lax.axis_index("c")
    lane = lax.iota(jnp.int32, _LANES)

    @pl.when(wid < _NTPA)
    def _pre_tasks():
        base = wid * _RPT
        pltpu.sync_copy(p_hbm.at[pl.ds(base, _RPT)], buf)
        accm = jnp.zeros((_LANES,), jnp.float32)
        accs = jnp.zeros((_LANES,), jnp.float32)
        for r in range(_RPT):
            rmax, rsq = _row_stats(buf, r, True)
            accm = jnp.where(lane == r, rmax, accm)
            accs = jnp.where(lane == r, rsq, accs)
        outa[...] = accm
        outb[...] = accs
        pltpu.sync_copy(outa.at[pl.ds(0, _RPT)],
                        stats_out.at[0, pl.ds(base, _RPT)])
        pltpu.sync_copy(outb.at[pl.ds(0, _RPT)],
                        stats_out.at[3, pl.ds(base, _RPT)])

    @pl.when(jnp.logical_and(wid >= _NTPA, wid < 2 * _NTPA))
    def _oh_tasks():
        base = (wid - _NTPA) * _RPT
        pltpu.sync_copy(oh_hbm.at[pl.ds(base, _RPT)], buf)
        accm = jnp.zeros((_LANES,), jnp.float32)
        for r in range(_RPT):
            rmax, _ = _row_stats(buf, r, False)
            accm = jnp.where(lane == r, rmax, accm)
        outa[...] = accm
        pltpu.sync_copy(outa.at[pl.ds(0, _RPT)],
                        stats_out.at[1, pl.ds(base, _RPT)])

    @pl.when(jnp.logical_and(wid >= 2 * _NTPA, wid < 3 * _NTPA))
    def _tf_tasks():
        base = (wid - 2 * _NTPA) * _RPT
        pltpu.sync_copy(tf_hbm.at[pl.ds(base, _RPT)], buf)
        accm = jnp.zeros((_LANES,), jnp.float32)
        for r in range(_RPT):
            rmax, _ = _row_stats(buf, r, False)
            accm = jnp.where(lane == r, rmax, accm)
        outa[...] = accm
        pltpu.sync_copy(outa.at[pl.ds(0, _RPT)],
                        stats_out.at[2, pl.ds(base, _RPT)])


_sc_rowstats = functools.partial(
    pl.kernel,
    out_type=jax.ShapeDtypeStruct((4, _B), jnp.float32),
    mesh=plsc.VectorSubcoreMesh(core_axis_name="c", subcore_axis_name="s"),
    scratch_types=[
        pltpu.VMEM((_RPT, _T), jnp.float32),
        pltpu.VMEM((_LANES,), jnp.float32),
        pltpu.VMEM((_LANES,), jnp.float32),
    ],
)(_sc_rowstats_body)


def _mpp_kernel(anchors_ref, variances_ref, sn_ref, sa_ref, out_ref, acc_ref):
    l = pl.program_id(0)
    kb = pl.program_id(1)

    @pl.when(jnp.logical_and(l == 0, kb == 0))
    def _init():
        acc_ref[0] = 0.0

    x = sn_ref[0]
    y = sa_ref[0]
    mu = anchors_ref[0]
    inv_var = 1.0 / variances_ref[0]
    dx = x - mu
    dy = y - mu
    d_pos = jnp.sqrt(jnp.sum(dx * dx * inv_var, axis=1, keepdims=True))
    d_neg = jnp.sqrt(jnp.sum(dy * dy * inv_var, axis=1, keepdims=True))
    acc_ref[0] += jnp.sum(jnp.maximum(d_pos - d_neg + 1.0, 0.0))

    @pl.when(jnp.logical_and(l == _L - 1, kb == _K // _BK - 1))
    def _finish():
        out_ref[0] = acc_ref[0] / _K


def _epilogue_kernel(stats_ref, mpp_ref, out_ref):
    an = stats_ref[0:1, :]
    ohm = stats_ref[1:2, :]
    tfm = stats_ref[2:3, :] * 2.5
    ssq = stats_ref[3:4, :]

    omax = jnp.max(ohm)
    omin = jnp.min(ohm)
    oh = jnp.where(omax > 1.0, (ohm - omin) / (omax - omin), ohm)
    tmax = jnp.max(tfm)
    tmin = jnp.min(tfm)
    tf = jnp.where(tmax > 1.0, (tfm - tmin) / (tmax - tmin), tfm)

    hp = jnp.maximum(oh, tf)
    hp_loss = jnp.mean((hp - an) ** 2)
    normal_loss = jnp.mean(jnp.sqrt(ssq))
    mpp_loss = mpp_ref[0]
    total_loss = normal_loss + mpp_loss
    out_ref[0] = 0.9 * total_loss + hp_loss
    out_ref[1] = normal_loss
    out_ref[2] = mpp_loss
    out_ref[3] = total_loss


def kernel(pre_normal_scores, oh_att, tf_att, anchors, variances,
           select_normals, select_abnormals):
    stats = _sc_rowstats(pre_normal_scores, oh_att, tf_att)

    mpp = pl.pallas_call(
        _mpp_kernel,
        grid=(_L, _K // _BK),
        in_specs=[
            pl.BlockSpec((1, 1, _C), lambda l, kb: (l, 0, 0)),
            pl.BlockSpec((1, 1, _C), lambda l, kb: (l, 0, 0)),
            pl.BlockSpec((1, _BK, _C), lambda l, kb: (l, kb, 0)),
            pl.BlockSpec((1, _BK, _C), lambda l, kb: (l, kb, 0)),
        ],
        out_specs=pl.BlockSpec(memory_space=pltpu.SMEM),
        out_shape=jax.ShapeDtypeStruct((1,), jnp.float32),
        scratch_shapes=[pltpu.SMEM((1,), jnp.float32)],
    )(anchors.reshape(_L, 1, _C), variances.reshape(_L, 1, _C),
      select_normals, select_abnormals)

    out = pl.pallas_call(
        _epilogue_kernel,
        in_specs=[
            pl.BlockSpec((4, _B), lambda: (0, 0)),
            pl.BlockSpec(memory_space=pltpu.SMEM),
        ],
        out_specs=pl.BlockSpec(memory_space=pltpu.SMEM),
        out_shape=jax.ShapeDtypeStruct((4,), jnp.float32),
    )(stats, mpp)

    return out[0], out[1], out[2], out[3]


# speedup vs baseline: 11.0264x; 1.1409x over previous
import functools

import jax
import jax.numpy as jnp
from jax import lax
from jax.experimental import pallas as pl
from jax.experimental.pallas import tpu as pltpu
from jax.experimental.pallas import tpu_sc as plsc

_L, _K, _C = 2, 2048, 1024
_B, _T = 64, 4096
_BK = 512
_NC, _NS, _LANES = 2, 16, 16
_RPT = 8
_NTPA = _B // _RPT
_UNROLL = 4


def _lane_reduce(v, op):
    idx = lax.iota(jnp.int32, _LANES)
    for s in (8, 4, 2, 1):
        v = op(v, v.at[idx ^ s].get(mode="promise_in_bounds"))
    return v


def _row_stats(buf, r, with_sq):
    def body(i, carry):
        ms, ss = carry
        new_ms, new_ss = [], []
        for u in range(_UNROLL):
            v = buf[r, pl.ds((i * _UNROLL + u) * _LANES, _LANES)]
            new_ms.append(jnp.maximum(ms[u], v))
            if with_sq:
                new_ss.append(ss[u] + v * v)
        return tuple(new_ms), (tuple(new_ss) if with_sq else ss)

    neg = jnp.full((_LANES,), -jnp.inf, jnp.float32)
    zero = jnp.zeros((_LANES,), jnp.float32)
    m0 = (neg,) * _UNROLL
    s0 = (zero,) * _UNROLL if with_sq else ()
    m, s = lax.fori_loop(0, _T // (_UNROLL * _LANES), body, (m0, s0))
    mv = jnp.maximum(jnp.maximum(m[0], m[1]), jnp.maximum(m[2], m[3]))
    rmax = _lane_reduce(mv, jnp.maximum)
    if with_sq:
        rsq = _lane_reduce((s[0] + s[1]) + (s[2] + s[3]), jnp.add)
        return rmax, rsq
    return rmax, None


def _sc_rowstats_body(p_hbm, oh_hbm, tf_hbm, stats_out, buf, outa, outb):
    wid = lax.axis_index("s") * _NC + lax.axis_index("c")
    lane = lax.iota(jnp.int32, _LANES)

    @pl.when(wid < _NTPA)
    def _pre_tasks():
        base = wid * _RPT
        pltpu.sync_copy(p_hbm.at[pl.ds(base, _RPT)], buf)
        accm = jnp.zeros((_LANES,), jnp.float32)
        accs = jnp.zeros((_LANES,), jnp.float32)
        for r in range(_RPT):
            rmax, rsq = _row_stats(buf, r, True)
            accm = jnp.where(lane == r, rmax, accm)
            accs = jnp.where(lane == r, rsq, accs)
        outa[...] = accm
        outb[...] = accs
        pltpu.sync_copy(outa.at[pl.ds(0, _RPT)],
                        stats_out.at[0, pl.ds(base, _RPT)])
        pltpu.sync_copy(outb.at[pl.ds(0, _RPT)],
                        stats_out.at[3, pl.ds(base, _RPT)])

    @pl.when(jnp.logical_and(wid >= _NTPA, wid < 2 * _NTPA))
    def _oh_tasks():
        base = (wid - _NTPA) * _RPT
        pltpu.sync_copy(oh_hbm.at[pl.ds(base, _RPT)], buf)
        accm = jnp.zeros((_LANES,), jnp.float32)
        for r in range(_RPT):
            rmax, _ = _row_stats(buf, r, False)
            accm = jnp.where(lane == r, rmax, accm)
        outa[...] = accm
        pltpu.sync_copy(outa.at[pl.ds(0, _RPT)],
                        stats_out.at[1, pl.ds(base, _RPT)])

    @pl.when(jnp.logical_and(wid >= 2 * _NTPA, wid < 3 * _NTPA))
    def _tf_tasks():
        base = (wid - 2 * _NTPA) * _RPT
        pltpu.sync_copy(tf_hbm.at[pl.ds(base, _RPT)], buf)
        accm = jnp.zeros((_LANES,), jnp.float32)
        for r in range(_RPT):
            rmax, _ = _row_stats(buf, r, False)
            accm = jnp.where(lane == r, rmax, accm)
        outa[...] = accm
        pltpu.sync_copy(outa.at[pl.ds(0, _RPT)],
                        stats_out.at[2, pl.ds(base, _RPT)])


_sc_rowstats = functools.partial(
    pl.kernel,
    out_type=jax.ShapeDtypeStruct((4, _B), jnp.float32),
    mesh=plsc.VectorSubcoreMesh(core_axis_name="c", subcore_axis_name="s"),
    scratch_types=[
        pltpu.VMEM((_RPT, _T), jnp.float32),
        pltpu.VMEM((_LANES,), jnp.float32),
        pltpu.VMEM((_LANES,), jnp.float32),
    ],
)(_sc_rowstats_body)


def _mpp_kernel(anchors_ref, variances_ref, sn_ref, sa_ref, out_ref, acc_ref):
    kb = pl.program_id(0)

    @pl.when(kb == 0)
    def _init():
        acc_ref[0] = 0.0

    part = jnp.float32(0.0)
    for l in range(_L):
        x = sn_ref[l]
        y = sa_ref[l]
        mu = anchors_ref[l:l + 1]
        inv_var = 1.0 / variances_ref[l:l + 1]
        dx = x - mu
        dy = y - mu
        d_pos = jnp.sqrt(jnp.sum(dx * dx * inv_var, axis=1, keepdims=True))
        d_neg = jnp.sqrt(jnp.sum(dy * dy * inv_var, axis=1, keepdims=True))
        part = part + jnp.sum(jnp.maximum(d_pos - d_neg + 1.0, 0.0))
    acc_ref[0] += part

    @pl.when(kb == _K // _BK - 1)
    def _finish():
        out_ref[0] = acc_ref[0] / _K


def _epilogue_kernel(stats_ref, mpp_ref, *out_ref):
    an = stats_ref[0:1, :]
    ohm = stats_ref[1:2, :]
    tfm = stats_ref[2:3, :] * 2.5
    ssq = stats_ref[3:4, :]

    omax = jnp.max(ohm)
    omin = jnp.min(ohm)
    oh = jnp.where(omax > 1.0, (ohm - omin) / (omax - omin), ohm)
    tmax = jnp.max(tfm)
    tmin = jnp.min(tfm)
    tf = jnp.where(tmax > 1.0, (tfm - tmin) / (tmax - tmin), tfm)

    hp = jnp.maximum(oh, tf)
    hp_loss = jnp.mean((hp - an) ** 2)
    normal_loss = jnp.mean(jnp.sqrt(ssq))
    mpp_loss = mpp_ref[0]
    total_loss = normal_loss + mpp_loss
    nc_ref, nl_ref, mp_ref, tl_ref = out_ref
    nc_ref[0] = 0.9 * total_loss + hp_loss
    nl_ref[0] = normal_loss
    mp_ref[0] = mpp_loss
    tl_ref[0] = total_loss


def kernel(pre_normal_scores, oh_att, tf_att, anchors, variances,
           select_normals, select_abnormals):
    stats = _sc_rowstats(pre_normal_scores, oh_att, tf_att)

    mpp = pl.pallas_call(
        _mpp_kernel,
        grid=(_K // _BK,),
        in_specs=[
            pl.BlockSpec((_L, _C), lambda kb: (0, 0)),
            pl.BlockSpec((_L, _C), lambda kb: (0, 0)),
            pl.BlockSpec((_L, _BK, _C), lambda kb: (0, kb, 0)),
            pl.BlockSpec((_L, _BK, _C), lambda kb: (0, kb, 0)),
        ],
        out_specs=pl.BlockSpec(memory_space=pltpu.SMEM),
        out_shape=jax.ShapeDtypeStruct((1,), jnp.float32),
        scratch_shapes=[pltpu.SMEM((1,), jnp.float32)],
    )(anchors, variances, select_normals, select_abnormals)

    scalar = jax.ShapeDtypeStruct((1,), jnp.float32)
    out = pl.pallas_call(
        _epilogue_kernel,
        in_specs=[
            pl.BlockSpec((4, _B), lambda: (0, 0)),
            pl.BlockSpec(memory_space=pltpu.SMEM),
        ],
        out_specs=[pl.BlockSpec(memory_space=pltpu.SMEM)] * 4,
        out_shape=[scalar, scalar, scalar, scalar],
    )(stats, mpp)

    return tuple(o.reshape(()) for o in out)
